# baseline (speedup 1.0000x reference)
import functools

import jax
import jax.numpy as jnp
from jax import lax
from jax.experimental import pallas as pl
from jax.experimental.pallas import tpu as pltpu
from jax.experimental.pallas import tpu_sc as plsc

N = 4096
NC = 2
NS = 16
NW = NC * NS
RPW = 16
SC_ROWS = NW * RPW
RC = 8
NCHUNK = RPW // RC
NBUF = 2
L = 16
CPR = N // L

TC_BLK = 256
TC_ROWS = N - SC_ROWS


def _hsum_splat(v):
    idx = lax.iota(jnp.int32, L)
    for s in (8, 4, 2, 1):
        v = v + v.at[idx ^ s].get(mode="promise_in_bounds")
    return v


def _sc_body(A_hbm, x_hbm, b_hbm, y_hbm, out_hbm,
             x_v, b_v, y_v, buf0, buf1, part_v, acc_v, shared,
             sem0, sem1, semx):
    cid = lax.axis_index("c")
    sid = lax.axis_index("s")
    wid = cid * NS + sid
    row0 = wid * RPW

    pltpu.async_copy(x_hbm, x_v, semx).wait()
    pltpu.async_copy(b_hbm.at[pl.ds(row0, RPW)], b_v.at[pl.ds(0, RPW)],
                     semx).wait()
    pltpu.async_copy(y_hbm.at[pl.ds(row0, RPW)], y_v.at[pl.ds(0, RPW)],
                     semx).wait()

    bufs = (buf0, buf1)
    sems = (sem0, sem1)

    pltpu.async_copy(A_hbm.at[pl.ds(row0, RC), :], buf0, sem0)
    pltpu.async_copy(A_hbm.at[pl.ds(row0 + RC, RC), :], buf1, sem1)

    def do_chunk(g, buf, sem, total):
        pltpu.make_async_copy(A_hbm.at[pl.ds(0, RC), :], buf, sem).wait()

        def col_body(i, accs):
            xv = x_v[pl.ds(i * L, L)]
            return tuple(accs[r] + buf[r, pl.ds(i * L, L)] * xv
                         for r in range(RC))

        accs = plsc.parallel_loop(
            0, CPR, 1, unroll=2,
            carry=tuple(jnp.zeros((L,), jnp.float32) for _ in range(RC)),
        )(col_body)

        bvec = b_v[pl.ds(g * RC, L)]
        yvec = y_v[pl.ds(g * RC, L)]
        for r in range(RC):
            dvec = _hsum_splat(accs[r])
            total = total + jnp.abs((dvec - bvec[r]) * yvec[r])
        return total

    total = jnp.zeros((L,), jnp.float32)
    for g in range(NCHUNK):
        total = do_chunk(g, bufs[g % NBUF], sems[g % NBUF], total)

    part_v[...] = total
    pltpu.sync_copy(part_v, shared.at[pl.ds(sid * L, L)])
    plsc.subcore_barrier()

    @pl.when(sid == 0)
    def _():
        pltpu.sync_copy(shared, acc_v)
        tot = jnp.zeros((L,), jnp.float32)
        for r in range(NS):
            tot = tot + acc_v[pl.ds(r * L, L)]
        part_v[...] = tot
        pltpu.sync_copy(part_v, out_hbm.at[cid])


_launch_sc = functools.partial(
    pl.kernel,
    out_type=jax.ShapeDtypeStruct((NC, L), jnp.float32),
    mesh=plsc.VectorSubcoreMesh(core_axis_name="c", subcore_axis_name="s",
                                num_cores=NC, num_subcores=NS),
    scratch_types=[
        pltpu.VMEM((N,), jnp.float32),
        pltpu.VMEM((RPW + L,), jnp.float32),
        pltpu.VMEM((RPW + L,), jnp.float32),
        pltpu.VMEM((RC, N), jnp.float32),
        pltpu.VMEM((RC, N), jnp.float32),
        pltpu.VMEM((L,), jnp.float32),
        pltpu.VMEM((NS * L,), jnp.float32),
        pltpu.VMEM_SHARED((NS * L,), jnp.float32),
        pltpu.SemaphoreType.DMA,
        pltpu.SemaphoreType.DMA,
        pltpu.SemaphoreType.DMA,
    ],
)(_sc_body)


def _tc_body(A_ref, x_ref, b_ref, y_ref, out_ref):
    i = pl.program_id(0)
    ax = jnp.dot(A_ref[...], x_ref[...], preferred_element_type=jnp.float32)
    part = jnp.sum(jnp.abs((ax - b_ref[...]) * y_ref[...]))

    @pl.when(i == 0)
    def _():
        out_ref[0, 0] = 0.0

    out_ref[0, 0] += part


_launch_tc = pl.pallas_call(
    _tc_body,
    grid=(TC_ROWS // TC_BLK,),
    in_specs=[
        pl.BlockSpec((TC_BLK, N), lambda i: (i + SC_ROWS // TC_BLK, 0)),
        pl.BlockSpec((N, 1), lambda i: (0, 0)),
        pl.BlockSpec((TC_BLK, 1), lambda i: (i + SC_ROWS // TC_BLK, 0)),
        pl.BlockSpec((TC_BLK, 1), lambda i: (i + SC_ROWS // TC_BLK, 0)),
    ],
    out_specs=pl.BlockSpec(memory_space=pltpu.SMEM),
    out_shape=jax.ShapeDtypeStruct((1, 1), jnp.float32),
)


def kernel(Q, A, AT, b, c, x, y, il, iu, l, u):
    sc_out = _launch_sc(A, x.reshape(N), b, y.reshape(N))
    tc_out = _launch_tc(A, x, b.reshape(N, 1), y)
    return sc_out[0, 0] + sc_out[1, 0] + tc_out[0, 0]


# speedup vs baseline: 1.0523x; 1.0523x over previous
import functools

import jax
import jax.numpy as jnp
from jax import lax
from jax.experimental import pallas as pl
from jax.experimental.pallas import tpu as pltpu
from jax.experimental.pallas import tpu_sc as plsc

N = 4096
NC = 2
NS = 16
NW = NC * NS
RPW = 16
SC_ROWS = NW * RPW
RC = 8
NCHUNK = RPW // RC
NBUF = 2
L = 16
CPR = N // L

TC_BLK = 512
TC_ROWS = N - SC_ROWS


def _hsum_splat(v):
    idx = lax.iota(jnp.int32, L)
    for s in (8, 4, 2, 1):
        v = v + v.at[idx ^ s].get(mode="promise_in_bounds")
    return v


def _sc_body(A_hbm, x_hbm, b_hbm, y_hbm, out_hbm,
             x_v, b_v, y_v, buf0, buf1, part_v, acc_v, shared,
             sem0, sem1, semx):
    cid = lax.axis_index("c")
    sid = lax.axis_index("s")
    wid = cid * NS + sid
    row0 = wid * RPW

    pltpu.async_copy(x_hbm, x_v, semx).wait()
    pltpu.async_copy(b_hbm.at[pl.ds(row0, RPW)], b_v.at[pl.ds(0, RPW)],
                     semx).wait()
    pltpu.async_copy(y_hbm.at[pl.ds(row0, RPW)], y_v.at[pl.ds(0, RPW)],
                     semx).wait()

    bufs = (buf0, buf1)
    sems = (sem0, sem1)

    pltpu.async_copy(A_hbm.at[pl.ds(row0, RC), :], buf0, sem0)
    pltpu.async_copy(A_hbm.at[pl.ds(row0 + RC, RC), :], buf1, sem1)

    def do_chunk(g, buf, sem, total):
        pltpu.make_async_copy(A_hbm.at[pl.ds(0, RC), :], buf, sem).wait()

        def col_body(i, accs):
            xv = x_v[pl.ds(i * L, L)]
            return tuple(accs[r] + buf[r, pl.ds(i * L, L)] * xv
                         for r in range(RC))

        accs = plsc.parallel_loop(
            0, CPR, 1, unroll=2,
            carry=tuple(jnp.zeros((L,), jnp.float32) for _ in range(RC)),
        )(col_body)

        bvec = b_v[pl.ds(g * RC, L)]
        yvec = y_v[pl.ds(g * RC, L)]
        for r in range(RC):
            dvec = _hsum_splat(accs[r])
            total = total + jnp.abs((dvec - bvec[r]) * yvec[r])
        return total

    total = jnp.zeros((L,), jnp.float32)
    for g in range(NCHUNK):
        total = do_chunk(g, bufs[g % NBUF], sems[g % NBUF], total)

    part_v[...] = total
    pltpu.sync_copy(part_v, shared.at[pl.ds(sid * L, L)])
    plsc.subcore_barrier()

    @pl.when(sid == 0)
    def _():
        pltpu.sync_copy(shared, acc_v)
        tot = jnp.zeros((L,), jnp.float32)
        for r in range(NS):
            tot = tot + acc_v[pl.ds(r * L, L)]
        part_v[...] = tot
        pltpu.sync_copy(part_v, out_hbm.at[cid])


_launch_sc = functools.partial(
    pl.kernel,
    out_type=jax.ShapeDtypeStruct((NC, L), jnp.float32),
    mesh=plsc.VectorSubcoreMesh(core_axis_name="c", subcore_axis_name="s",
                                num_cores=NC, num_subcores=NS),
    scratch_types=[
        pltpu.VMEM((N,), jnp.float32),
        pltpu.VMEM((RPW + L,), jnp.float32),
        pltpu.VMEM((RPW + L,), jnp.float32),
        pltpu.VMEM((RC, N), jnp.float32),
        pltpu.VMEM((RC, N), jnp.float32),
        pltpu.VMEM((L,), jnp.float32),
        pltpu.VMEM((NS * L,), jnp.float32),
        pltpu.VMEM_SHARED((NS * L,), jnp.float32),
        pltpu.SemaphoreType.DMA,
        pltpu.SemaphoreType.DMA,
        pltpu.SemaphoreType.DMA,
    ],
)(_sc_body)


def _tc_body(A_ref, x_ref, b_ref, y_ref, out_ref):
    i = pl.program_id(0)
    ax = jnp.dot(A_ref[...], x_ref[...], preferred_element_type=jnp.float32)
    part = jnp.sum(jnp.abs((ax - b_ref[...]) * y_ref[...]))

    @pl.when(i == 0)
    def _():
        out_ref[0, 0] = 0.0

    out_ref[0, 0] += part


_launch_tc = pl.pallas_call(
    _tc_body,
    grid=(TC_ROWS // TC_BLK,),
    in_specs=[
        pl.BlockSpec((TC_BLK, N), lambda i: (i + SC_ROWS // TC_BLK, 0)),
        pl.BlockSpec((N, 1), lambda i: (0, 0)),
        pl.BlockSpec((TC_BLK, 1), lambda i: (i + SC_ROWS // TC_BLK, 0)),
        pl.BlockSpec((TC_BLK, 1), lambda i: (i + SC_ROWS // TC_BLK, 0)),
    ],
    out_specs=pl.BlockSpec(memory_space=pltpu.SMEM),
    out_shape=jax.ShapeDtypeStruct((1, 1), jnp.float32),
)


def kernel(Q, A, AT, b, c, x, y, il, iu, l, u):
    sc_out = _launch_sc(A, x.reshape(N), b, y.reshape(N))
    tc_out = _launch_tc(A, x, b.reshape(N, 1), y)
    return sc_out[0, 0] + sc_out[1, 0] + tc_out[0, 0]


# speedup vs baseline: 1.0636x; 1.0107x over previous
import functools

import jax
import jax.numpy as jnp
from jax import lax
from jax.experimental import pallas as pl
from jax.experimental.pallas import tpu as pltpu
from jax.experimental.pallas import tpu_sc as plsc

N = 4096
NC = 2
NS = 16
NW = NC * NS
RPW = 32
SC_ROWS = NW * RPW
RC = 8
NCHUNK = RPW // RC
NBUF = 2
L = 16
CPR = N // L

TC_BLK = 512
TC_ROWS = N - SC_ROWS


def _hsum_splat(v):
    idx = lax.iota(jnp.int32, L)
    for s in (8, 4, 2, 1):
        v = v + v.at[idx ^ s].get(mode="promise_in_bounds")
    return v


def _sc_body(A_hbm, x_hbm, b_hbm, y_hbm, out_hbm,
             x_v, b_v, y_v, buf0, buf1, part_v, acc_v, shared,
             sem0, sem1, semx):
    cid = lax.axis_index("c")
    sid = lax.axis_index("s")
    wid = cid * NS + sid
    row0 = wid * RPW

    pltpu.async_copy(x_hbm, x_v, semx).wait()
    pltpu.async_copy(b_hbm.at[pl.ds(row0, RPW)], b_v.at[pl.ds(0, RPW)],
                     semx).wait()
    pltpu.async_copy(y_hbm.at[pl.ds(row0, RPW)], y_v.at[pl.ds(0, RPW)],
                     semx).wait()

    bufs = (buf0, buf1)
    sems = (sem0, sem1)

    pltpu.async_copy(A_hbm.at[pl.ds(row0, RC), :], buf0, sem0)
    pltpu.async_copy(A_hbm.at[pl.ds(row0 + RC, RC), :], buf1, sem1)

    def do_chunk(g, buf, sem, total):
        pltpu.make_async_copy(A_hbm.at[pl.ds(0, RC), :], buf, sem).wait()

        def col_body(i, accs):
            xv = x_v[pl.ds(i * L, L)]
            return tuple(accs[r] + buf[r, pl.ds(i * L, L)] * xv
                         for r in range(RC))

        accs = plsc.parallel_loop(
            0, CPR, 1, unroll=2,
            carry=tuple(jnp.zeros((L,), jnp.float32) for _ in range(RC)),
        )(col_body)

        if g + NBUF < NCHUNK:
            pltpu.async_copy(
                A_hbm.at[pl.ds(row0 + (g + NBUF) * RC, RC), :], buf, sem)

        bvec = b_v[pl.ds(g * RC, L)]
        yvec = y_v[pl.ds(g * RC, L)]
        for r in range(RC):
            dvec = _hsum_splat(accs[r])
            total = total + jnp.abs((dvec - bvec[r]) * yvec[r])
        return total

    total = jnp.zeros((L,), jnp.float32)
    for g in range(NCHUNK):
        total = do_chunk(g, bufs[g % NBUF], sems[g % NBUF], total)

    part_v[...] = total
    pltpu.sync_copy(part_v, shared.at[pl.ds(sid * L, L)])
    plsc.subcore_barrier()

    @pl.when(sid == 0)
    def _():
        pltpu.sync_copy(shared, acc_v)
        tot = jnp.zeros((L,), jnp.float32)
        for r in range(NS):
            tot = tot + acc_v[pl.ds(r * L, L)]
        part_v[...] = tot
        pltpu.sync_copy(part_v, out_hbm.at[cid])


_launch_sc = functools.partial(
    pl.kernel,
    out_type=jax.ShapeDtypeStruct((NC, L), jnp.float32),
    mesh=plsc.VectorSubcoreMesh(core_axis_name="c", subcore_axis_name="s",
                                num_cores=NC, num_subcores=NS),
    scratch_types=[
        pltpu.VMEM((N,), jnp.float32),
        pltpu.VMEM((RPW + L,), jnp.float32),
        pltpu.VMEM((RPW + L,), jnp.float32),
        pltpu.VMEM((RC, N), jnp.float32),
        pltpu.VMEM((RC, N), jnp.float32),
        pltpu.VMEM((L,), jnp.float32),
        pltpu.VMEM((NS * L,), jnp.float32),
        pltpu.VMEM_SHARED((NS * L,), jnp.float32),
        pltpu.SemaphoreType.DMA,
        pltpu.SemaphoreType.DMA,
        pltpu.SemaphoreType.DMA,
    ],
)(_sc_body)


def _tc_body(A_ref, x_ref, b_ref, y_ref, out_ref):
    i = pl.program_id(0)
    ax = jnp.dot(A_ref[...], x_ref[...], preferred_element_type=jnp.float32)
    part = jnp.sum(jnp.abs((ax - b_ref[...]) * y_ref[...]))

    @pl.when(i == 0)
    def _():
        out_ref[0, 0] = 0.0

    out_ref[0, 0] += part


_launch_tc = pl.pallas_call(
    _tc_body,
    grid=(TC_ROWS // TC_BLK,),
    in_specs=[
        pl.BlockSpec((TC_BLK, N), lambda i: (i + SC_ROWS // TC_BLK, 0)),
        pl.BlockSpec((N, 1), lambda i: (0, 0)),
        pl.BlockSpec((TC_BLK, 1), lambda i: (i + SC_ROWS // TC_BLK, 0)),
        pl.BlockSpec((TC_BLK, 1), lambda i: (i + SC_ROWS // TC_BLK, 0)),
    ],
    out_specs=pl.BlockSpec(memory_space=pltpu.SMEM),
    out_shape=jax.ShapeDtypeStruct((1, 1), jnp.float32),
)


def kernel(Q, A, AT, b, c, x, y, il, iu, l, u):
    sc_out = _launch_sc(A, x.reshape(N), b, y.reshape(N))
    tc_out = _launch_tc(A, x, b.reshape(N, 1), y)
    return sc_out[0, 0] + sc_out[1, 0] + tc_out[0, 0]


# speedup vs baseline: 1.0761x; 1.0118x over previous
import functools

import jax
import jax.numpy as jnp
from jax import lax
from jax.experimental import pallas as pl
from jax.experimental.pallas import tpu as pltpu
from jax.experimental.pallas import tpu_sc as plsc

N = 4096
NC = 2
NS = 16
NW = NC * NS
RPW = 64
SC_ROWS = NW * RPW
RC = 8
NCHUNK = RPW // RC
NBUF = 2
L = 16
CPR = N // L

TC_BLK = 512
TC_ROWS = N - SC_ROWS


def _hsum_splat(v):
    idx = lax.iota(jnp.int32, L)
    for s in (8, 4, 2, 1):
        v = v + v.at[idx ^ s].get(mode="promise_in_bounds")
    return v


def _sc_body(A_hbm, x_hbm, b_hbm, y_hbm, out_hbm,
             x_v, b_v, y_v, buf0, buf1, part_v, acc_v, shared,
             sem0, sem1, semx):
    cid = lax.axis_index("c")
    sid = lax.axis_index("s")
    wid = cid * NS + sid
    row0 = wid * RPW

    pltpu.async_copy(x_hbm, x_v, semx).wait()
    pltpu.async_copy(b_hbm.at[pl.ds(row0, RPW)], b_v.at[pl.ds(0, RPW)],
                     semx).wait()
    pltpu.async_copy(y_hbm.at[pl.ds(row0, RPW)], y_v.at[pl.ds(0, RPW)],
                     semx).wait()

    bufs = (buf0, buf1)
    sems = (sem0, sem1)

    pltpu.async_copy(A_hbm.at[pl.ds(row0, RC), :], buf0, sem0)
    pltpu.async_copy(A_hbm.at[pl.ds(row0 + RC, RC), :], buf1, sem1)

    def do_chunk(g, buf, sem, total):
        pltpu.make_async_copy(A_hbm.at[pl.ds(0, RC), :], buf, sem).wait()

        def col_body(i, accs):
            xv = x_v[pl.ds(i * L, L)]
            return tuple(accs[r] + buf[r, pl.ds(i * L, L)] * xv
                         for r in range(RC))

        accs = plsc.parallel_loop(
            0, CPR, 1, unroll=2,
            carry=tuple(jnp.zeros((L,), jnp.float32) for _ in range(RC)),
        )(col_body)

        if g + NBUF < NCHUNK:
            pltpu.async_copy(
                A_hbm.at[pl.ds(row0 + (g + NBUF) * RC, RC), :], buf, sem)

        bvec = b_v[pl.ds(g * RC, L)]
        yvec = y_v[pl.ds(g * RC, L)]
        for r in range(RC):
            dvec = _hsum_splat(accs[r])
            total = total + jnp.abs((dvec - bvec[r]) * yvec[r])
        return total

    total = jnp.zeros((L,), jnp.float32)
    for g in range(NCHUNK):
        total = do_chunk(g, bufs[g % NBUF], sems[g % NBUF], total)

    part_v[...] = total
    pltpu.sync_copy(part_v, shared.at[pl.ds(sid * L, L)])
    plsc.subcore_barrier()

    @pl.when(sid == 0)
    def _():
        pltpu.sync_copy(shared, acc_v)
        tot = jnp.zeros((L,), jnp.float32)
        for r in range(NS):
            tot = tot + acc_v[pl.ds(r * L, L)]
        part_v[...] = tot
        pltpu.sync_copy(part_v, out_hbm.at[cid])


_launch_sc = functools.partial(
    pl.kernel,
    out_type=jax.ShapeDtypeStruct((NC, L), jnp.float32),
    mesh=plsc.VectorSubcoreMesh(core_axis_name="c", subcore_axis_name="s",
                                num_cores=NC, num_subcores=NS),
    scratch_types=[
        pltpu.VMEM((N,), jnp.float32),
        pltpu.VMEM((RPW + L,), jnp.float32),
        pltpu.VMEM((RPW + L,), jnp.float32),
        pltpu.VMEM((RC, N), jnp.float32),
        pltpu.VMEM((RC, N), jnp.float32),
        pltpu.VMEM((L,), jnp.float32),
        pltpu.VMEM((NS * L,), jnp.float32),
        pltpu.VMEM_SHARED((NS * L,), jnp.float32),
        pltpu.SemaphoreType.DMA,
        pltpu.SemaphoreType.DMA,
        pltpu.SemaphoreType.DMA,
    ],
)(_sc_body)


def _tc_body(A_ref, x_ref, b_ref, y_ref, out_ref):
    i = pl.program_id(0)
    ax = jnp.dot(A_ref[...], x_ref[...], preferred_element_type=jnp.float32)
    part = jnp.sum(jnp.abs((ax - b_ref[...]) * y_ref[...]))

    @pl.when(i == 0)
    def _():
        out_ref[0, 0] = 0.0

    out_ref[0, 0] += part


_launch_tc = pl.pallas_call(
    _tc_body,
    grid=(TC_ROWS // TC_BLK,),
    in_specs=[
        pl.BlockSpec((TC_BLK, N), lambda i: (i + SC_ROWS // TC_BLK, 0)),
        pl.BlockSpec((N, 1), lambda i: (0, 0)),
        pl.BlockSpec((TC_BLK, 1), lambda i: (i + SC_ROWS // TC_BLK, 0)),
        pl.BlockSpec((TC_BLK, 1), lambda i: (i + SC_ROWS // TC_BLK, 0)),
    ],
    out_specs=pl.BlockSpec(memory_space=pltpu.SMEM),
    out_shape=jax.ShapeDtypeStruct((1, 1), jnp.float32),
)


def kernel(Q, A, AT, b, c, x, y, il, iu, l, u):
    sc_out = _launch_sc(A, x.reshape(N), b, y.reshape(N))
    tc_out = _launch_tc(A, x, b.reshape(N, 1), y)
    return sc_out[0, 0] + sc_out[1, 0] + tc_out[0, 0]
